# initial kernel scaffold (unmeasured)
import jax
import jax.numpy as jnp
from jax import lax
from jax.experimental import pallas as pl
from jax.experimental.pallas import tpu as pltpu


def kernel(
    x,
):
    def body(*refs):
        pass

    out_shape = jax.ShapeDtypeStruct(..., jnp.float32)
    return pl.pallas_call(body, out_shape=out_shape)(...)


# baseline (device time: 38598 ns/iter reference)
import jax
import jax.numpy as jnp
from jax import lax
from jax.experimental import pallas as pl
from jax.experimental.pallas import tpu as pltpu

N_Y = 4
K = 16


def _topk_desc(v, k, axes):
    cols = []
    for _ in range(k):
        m = jnp.max(v, axis=axes)
        cols.append(m)
        if len(axes) == 1:
            mb = m[:, None]
        else:
            mb = m[None, :, None]
        v = jnp.where(v == mb, -jnp.inf, v)
    return cols


def kernel(x):
    m_rows, n_cols = x.shape

    def body(x_ref, out_ref, cand_ref, send_sems, recv_sems):
        my_x = lax.axis_index("x")
        my_y = lax.axis_index("y")
        my_z = lax.axis_index("z")

        v = x_ref[...]
        local = jnp.stack(_topk_desc(v, K, (1,)), axis=1)
        cand_ref[my_y] = local

        barrier_sem = pltpu.get_barrier_semaphore()
        for t in range(N_Y):
            @pl.when(t != my_y)
            def _():
                pl.semaphore_signal(
                    barrier_sem, inc=1,
                    device_id=(my_x, t, my_z),
                    device_id_type=pl.DeviceIdType.MESH,
                )
        pl.semaphore_wait(barrier_sem, N_Y - 1)

        for t in range(N_Y):
            @pl.when(t != my_y)
            def _():
                rdma = pltpu.make_async_remote_copy(
                    src_ref=cand_ref.at[my_y],
                    dst_ref=cand_ref.at[my_y],
                    send_sem=send_sems.at[t],
                    recv_sem=recv_sems.at[my_y],
                    device_id=(my_x, t, my_z),
                    device_id_type=pl.DeviceIdType.MESH,
                )
                rdma.start()
                rdma.wait_send()

        for s in range(N_Y):
            @pl.when(s != my_y)
            def _():
                recv = pltpu.make_async_remote_copy(
                    src_ref=cand_ref.at[s],
                    dst_ref=cand_ref.at[s],
                    send_sem=send_sems.at[s],
                    recv_sem=recv_sems.at[s],
                    device_id=(my_x, my_y, my_z),
                    device_id_type=pl.DeviceIdType.MESH,
                )
                recv.wait_recv()

        w = cand_ref[...]
        out_ref[...] = jnp.stack(_topk_desc(w, K, (0, 2)), axis=1)

    return pl.pallas_call(
        body,
        out_shape=jax.ShapeDtypeStruct((m_rows, K), jnp.float32),
        in_specs=[pl.BlockSpec(memory_space=pltpu.VMEM)],
        out_specs=pl.BlockSpec(memory_space=pltpu.VMEM),
        scratch_shapes=[
            pltpu.VMEM((N_Y, m_rows, K), jnp.float32),
            pltpu.SemaphoreType.DMA((N_Y,)),
            pltpu.SemaphoreType.DMA((N_Y,)),
        ],
        compiler_params=pltpu.CompilerParams(collective_id=0),
    )(x)


# device time: 29837 ns/iter; 1.2936x vs baseline; 1.2936x over previous
import jax
import jax.numpy as jnp
from jax import lax
from jax.experimental import pallas as pl
from jax.experimental.pallas import tpu as pltpu

N_Y = 4
K = 16
N_SLICES = 16
DEPTH = 6


def _masked_max_topk(v, k):
    cols = []
    for _ in range(k):
        m = jnp.max(v, axis=1)
        cols.append(m)
        v = jnp.where(v == m[:, None], -jnp.inf, v)
    return jnp.stack(cols, axis=1), cols[-1]


def kernel(x):
    m_rows, n_cols = x.shape
    sl = n_cols // N_SLICES

    def body(x_ref, out_ref, cand_ref, send_sems, recv_sems):
        my_x = lax.axis_index("x")
        my_y = lax.axis_index("y")
        my_z = lax.axis_index("z")

        barrier_sem = pltpu.get_barrier_semaphore()
        for t in range(N_Y):
            @pl.when(t != my_y)
            def _():
                pl.semaphore_signal(
                    barrier_sem, inc=1,
                    device_id=(my_x, t, my_z),
                    device_id_type=pl.DeviceIdType.MESH,
                )

        ts = []
        for j in range(N_SLICES):
            cur = x_ref[:, j * sl:(j + 1) * sl]
            for i in range(len(ts)):
                hi = jnp.maximum(ts[i], cur)
                cur = jnp.minimum(ts[i], cur)
                ts[i] = hi
            if len(ts) < DEPTH:
                ts.append(cur)
        t6 = ts[DEPTH - 1]

        fs, tstar = _masked_max_topk(jnp.concatenate(ts[:DEPTH - 1], axis=1), K)

        pred = jnp.any(t6 >= tstar[:, None])

        @pl.when(jnp.logical_not(pred))
        def _():
            cand_ref[my_y] = fs

        @pl.when(pred)
        def _():
            full, _ = _masked_max_topk(x_ref[...], K)
            cand_ref[my_y] = full

        pl.semaphore_wait(barrier_sem, N_Y - 1)

        for t in range(N_Y):
            @pl.when(t != my_y)
            def _():
                rdma = pltpu.make_async_remote_copy(
                    src_ref=cand_ref.at[my_y],
                    dst_ref=cand_ref.at[my_y],
                    send_sem=send_sems.at[t],
                    recv_sem=recv_sems.at[my_y],
                    device_id=(my_x, t, my_z),
                    device_id_type=pl.DeviceIdType.MESH,
                )
                rdma.start()
                rdma.wait_send()

        for s in range(N_Y):
            @pl.when(s != my_y)
            def _():
                recv = pltpu.make_async_remote_copy(
                    src_ref=cand_ref.at[s],
                    dst_ref=cand_ref.at[s],
                    send_sem=send_sems.at[s],
                    recv_sem=recv_sems.at[s],
                    device_id=(my_x, my_y, my_z),
                    device_id_type=pl.DeviceIdType.MESH,
                )
                recv.wait_recv()

        w = cand_ref[...]
        cols = []
        for _ in range(K):
            m = jnp.max(w, axis=(0, 2))
            cols.append(m)
            w = jnp.where(w == m[None, :, None], -jnp.inf, w)
        out_ref[...] = jnp.stack(cols, axis=1)

    return pl.pallas_call(
        body,
        out_shape=jax.ShapeDtypeStruct((m_rows, K), jnp.float32),
        in_specs=[pl.BlockSpec(memory_space=pltpu.VMEM)],
        out_specs=pl.BlockSpec(memory_space=pltpu.VMEM),
        scratch_shapes=[
            pltpu.VMEM((N_Y, m_rows, K), jnp.float32),
            pltpu.SemaphoreType.DMA((N_Y,)),
            pltpu.SemaphoreType.DMA((N_Y,)),
        ],
        compiler_params=pltpu.CompilerParams(collective_id=0),
    )(x)
